# Initial kernel scaffold; baseline (speedup 1.0000x reference)
import jax
import jax.numpy as jnp
from jax.experimental import pallas as pl


def kernel(indices, tables):
    raise NotImplementedError("write your pallas kernel here")


# speedup vs baseline: 2.9534x; 2.9534x over previous
import functools

import jax
import jax.numpy as jnp
from jax import lax
from jax.experimental import pallas as pl
from jax.experimental.pallas import tpu as pltpu
from jax.experimental.pallas import tpu_sc as plsc

NUM_FIELDS = 26
VOCAB = 100000
EMBED_DIM = 16
BATCH = 4096
SEQ = 20
POSITIONS = BATCH * SEQ

_info = plsc.get_sparse_core_info()
NC = _info.num_cores
NS = _info.num_subcores
NW = NC * NS
B_PER_W = POSITIONS // NW


def _sc_body(idx_hbm, tab_hbm, out_hbm, idx_v, rows_v, sem):
    wid = lax.axis_index("s") * NC + lax.axis_index("c")
    base = wid * B_PER_W
    for f in range(NUM_FIELDS):
        pltpu.sync_copy(idx_hbm.at[f, pl.ds(base, B_PER_W)], idx_v)
        pltpu.async_copy(tab_hbm.at[f].at[idx_v], rows_v, sem).wait()
        pltpu.sync_copy(
            rows_v, out_hbm.at[pl.ds(base, B_PER_W), pl.ds(f * EMBED_DIM, EMBED_DIM)]
        )


_gather_kernel = pl.kernel(
    _sc_body,
    out_type=jax.ShapeDtypeStruct((POSITIONS, NUM_FIELDS * EMBED_DIM), jnp.float32),
    mesh=plsc.VectorSubcoreMesh(core_axis_name="c", subcore_axis_name="s"),
    scratch_types=[
        pltpu.VMEM((B_PER_W,), jnp.int32),
        pltpu.VMEM((B_PER_W, EMBED_DIM), jnp.float32),
        pltpu.SemaphoreType.DMA,
    ],
    compiler_params=pltpu.CompilerParams(use_tc_tiling_on_sc=False),
)


@jax.jit
def kernel(indices, tables):
    idx2 = indices.reshape(NUM_FIELDS, POSITIONS)
    out = _gather_kernel(idx2, tables)
    return out.reshape(BATCH, SEQ, NUM_FIELDS * EMBED_DIM)


# speedup vs baseline: 3.0391x; 1.0290x over previous
import functools

import jax
import jax.numpy as jnp
from jax import lax
from jax.experimental import pallas as pl
from jax.experimental.pallas import tpu as pltpu
from jax.experimental.pallas import tpu_sc as plsc

NUM_FIELDS = 26
VOCAB = 100000
EMBED_DIM = 16
BATCH = 4096
SEQ = 20
POSITIONS = BATCH * SEQ

_info = plsc.get_sparse_core_info()
NC = _info.num_cores
NS = _info.num_subcores
NW = NC * NS
B_PER_W = POSITIONS // NW


def _sc_body(idx_hbm, tab_hbm, out_hbm, idx_v, rows_v, sem_i, sem_g, sem_o):
    wid = lax.axis_index("s") * NC + lax.axis_index("c")
    base = wid * B_PER_W

    def start_idx(f):
        b = f % 2
        return pltpu.async_copy(
            idx_hbm.at[f, pl.ds(base, B_PER_W)], idx_v.at[b], sem_i.at[b]
        )

    idx_cp = start_idx(0)
    outs = [None, None]
    for f in range(NUM_FIELDS):
        b = f % 2
        next_idx = start_idx(f + 1) if f + 1 < NUM_FIELDS else None
        idx_cp.wait()
        if outs[b] is not None:
            outs[b].wait()
        g = pltpu.async_copy(tab_hbm.at[f].at[idx_v.at[b]], rows_v.at[b], sem_g.at[b])
        g.wait()
        outs[b] = pltpu.async_copy(
            rows_v.at[b],
            out_hbm.at[pl.ds(base, B_PER_W), pl.ds(f * EMBED_DIM, EMBED_DIM)],
            sem_o.at[b],
        )
        idx_cp = next_idx
    outs[0].wait()
    outs[1].wait()


_gather_kernel = pl.kernel(
    _sc_body,
    out_type=jax.ShapeDtypeStruct((POSITIONS, NUM_FIELDS * EMBED_DIM), jnp.float32),
    mesh=plsc.VectorSubcoreMesh(core_axis_name="c", subcore_axis_name="s"),
    scratch_types=[
        pltpu.VMEM((2, B_PER_W), jnp.int32),
        pltpu.VMEM((2, B_PER_W, EMBED_DIM), jnp.float32),
        pltpu.SemaphoreType.DMA((2,)),
        pltpu.SemaphoreType.DMA((2,)),
        pltpu.SemaphoreType.DMA((2,)),
    ],
    compiler_params=pltpu.CompilerParams(use_tc_tiling_on_sc=False),
)


@jax.jit
def kernel(indices, tables):
    idx2 = indices.reshape(NUM_FIELDS, POSITIONS)
    out = _gather_kernel(idx2, tables)
    return out.reshape(BATCH, SEQ, NUM_FIELDS * EMBED_DIM)
